# baseline (speedup 1.0000x reference)
import functools

import jax
import jax.numpy as jnp
from jax.experimental import pallas as pl
from jax.experimental.pallas import tpu as pltpu

HID = 768
HEADS = 12
DH = 64
NE = 8
TOP2 = 2
FFN = 768
SEQ = 2048
BLK = 128
NB = 40
ROWB = 256
QB = 512


def _ln_qkv_kernel(x_ref, g_ref, b_ref, w_ref, bias_ref, o_ref):
    x = x_ref[...]
    m = jnp.mean(x, axis=-1, keepdims=True)
    v = jnp.mean(jnp.square(x - m), axis=-1, keepdims=True)
    xn = (x - m) * jax.lax.rsqrt(v + 1e-5) * g_ref[...] + b_ref[...]
    o_ref[...] = jnp.dot(xn, w_ref[...],
                         preferred_element_type=jnp.float32) + bias_ref[...]


def _attn_kernel(q_ref, k_ref, v_ref, o_ref):
    q = q_ref[0]
    k = k_ref[0]
    v = v_ref[0]
    s = jax.lax.dot_general(q, k, (((1,), (1,)), ((), ())),
                            preferred_element_type=jnp.float32) * (DH ** -0.5)
    m = jnp.max(s, axis=-1, keepdims=True)
    p = jnp.exp(s - m)
    l = jnp.sum(p, axis=-1, keepdims=True)
    o = jnp.dot(p, v, preferred_element_type=jnp.float32) / l
    o_ref[0] = o


def _proj_ln2_route_kernel(a_ref, wo_ref, bo_ref, res_ref, g2_ref, b2_ref,
                           wr_ref, br_ref, h1_ref, t_ref, gates_ref, idx_ref):
    a = a_ref[...]
    h1 = jnp.dot(a, wo_ref[...],
                 preferred_element_type=jnp.float32) + bo_ref[...] + res_ref[...]
    h1_ref[...] = h1
    m = jnp.mean(h1, axis=-1, keepdims=True)
    v = jnp.mean(jnp.square(h1 - m), axis=-1, keepdims=True)
    t = (h1 - m) * jax.lax.rsqrt(v + 1e-5) * g2_ref[...] + b2_ref[...]
    t_ref[...] = t
    logits = jnp.dot(t, wr_ref[...],
                     preferred_element_type=jnp.float32) + br_ref[...]
    lm = jnp.max(logits, axis=-1, keepdims=True)
    pe = jnp.exp(logits - lm)
    probs = pe / jnp.sum(pe, axis=-1, keepdims=True)
    i1 = jnp.argmax(probs, axis=-1)
    v1 = jnp.max(probs, axis=-1)
    cols = jax.lax.broadcasted_iota(jnp.int32, probs.shape, 1)
    masked = jnp.where(cols == i1[:, None], -jnp.inf, probs)
    i2 = jnp.argmax(masked, axis=-1)
    v2 = jnp.max(masked, axis=-1)
    tot = v1 + v2
    gates_ref[:, 0] = v1 / tot
    gates_ref[:, 1] = v2 / tot
    idx_ref[:, 0] = i1.astype(jnp.int32)
    idx_ref[:, 1] = i2.astype(jnp.int32)


def _moe_kernel(be_ref, xg_ref, w1_ref, b1_ref, w2_ref, b2_ref, gate_ref,
                yp_ref):
    del be_ref
    xx = xg_ref[...]
    h = jnp.dot(xx, w1_ref[0], preferred_element_type=jnp.float32) + b1_ref[0]
    h = jax.nn.gelu(h)
    y = jnp.dot(h, w2_ref[0], preferred_element_type=jnp.float32) + b2_ref[0]
    yp_ref[...] = y * gate_ref[...]


def kernel(x, gamma1, beta1, Wqkv, bqkv, Wo, bo, gamma2, beta2, Wr, br,
           W1, b1, W2, b2):
    xf = x.reshape(SEQ, HID)

    qkv = pl.pallas_call(
        _ln_qkv_kernel,
        grid=(SEQ // ROWB,),
        in_specs=[
            pl.BlockSpec((ROWB, HID), lambda i: (i, 0)),
            pl.BlockSpec((1, HID), lambda i: (0, 0)),
            pl.BlockSpec((1, HID), lambda i: (0, 0)),
            pl.BlockSpec((HID, 3 * HID), lambda i: (0, 0)),
            pl.BlockSpec((1, 3 * HID), lambda i: (0, 0)),
        ],
        out_specs=pl.BlockSpec((ROWB, 3 * HID), lambda i: (i, 0)),
        out_shape=jax.ShapeDtypeStruct((SEQ, 3 * HID), jnp.float32),
        compiler_params=pltpu.CompilerParams(
            dimension_semantics=("arbitrary",)),
    )(xf, gamma1.reshape(1, HID), beta1.reshape(1, HID), Wqkv.T,
      bqkv.reshape(1, 3 * HID))

    q = qkv[:, :HID].reshape(SEQ, HEADS, DH).transpose(1, 0, 2)
    k = qkv[:, HID:2 * HID].reshape(SEQ, HEADS, DH).transpose(1, 0, 2)
    v = qkv[:, 2 * HID:].reshape(SEQ, HEADS, DH).transpose(1, 0, 2)

    attn = pl.pallas_call(
        _attn_kernel,
        grid=(HEADS, SEQ // QB),
        in_specs=[
            pl.BlockSpec((1, QB, DH), lambda h, i: (h, i, 0)),
            pl.BlockSpec((1, SEQ, DH), lambda h, i: (h, 0, 0)),
            pl.BlockSpec((1, SEQ, DH), lambda h, i: (h, 0, 0)),
        ],
        out_specs=pl.BlockSpec((1, QB, DH), lambda h, i: (h, i, 0)),
        out_shape=jax.ShapeDtypeStruct((HEADS, SEQ, DH), jnp.float32),
        compiler_params=pltpu.CompilerParams(
            dimension_semantics=("arbitrary", "arbitrary")),
    )(q, k, v)
    attn = attn.transpose(1, 0, 2).reshape(SEQ, HID)

    h1, t, gates, idx = pl.pallas_call(
        _proj_ln2_route_kernel,
        grid=(SEQ // ROWB,),
        in_specs=[
            pl.BlockSpec((ROWB, HID), lambda i: (i, 0)),
            pl.BlockSpec((HID, HID), lambda i: (0, 0)),
            pl.BlockSpec((1, HID), lambda i: (0, 0)),
            pl.BlockSpec((ROWB, HID), lambda i: (i, 0)),
            pl.BlockSpec((1, HID), lambda i: (0, 0)),
            pl.BlockSpec((1, HID), lambda i: (0, 0)),
            pl.BlockSpec((HID, NE), lambda i: (0, 0)),
            pl.BlockSpec((1, NE), lambda i: (0, 0)),
        ],
        out_specs=[
            pl.BlockSpec((ROWB, HID), lambda i: (i, 0)),
            pl.BlockSpec((ROWB, HID), lambda i: (i, 0)),
            pl.BlockSpec((ROWB, TOP2), lambda i: (i, 0)),
            pl.BlockSpec((ROWB, TOP2), lambda i: (i, 0)),
        ],
        out_shape=[
            jax.ShapeDtypeStruct((SEQ, HID), jnp.float32),
            jax.ShapeDtypeStruct((SEQ, HID), jnp.float32),
            jax.ShapeDtypeStruct((SEQ, TOP2), jnp.float32),
            jax.ShapeDtypeStruct((SEQ, TOP2), jnp.int32),
        ],
        compiler_params=pltpu.CompilerParams(
            dimension_semantics=("arbitrary",)),
    )(attn, Wo.T, bo.reshape(1, HID), xf, gamma2.reshape(1, HID),
      beta2.reshape(1, HID), Wr, br.reshape(1, NE))

    na = SEQ * TOP2
    eids = idx.reshape(na)
    gv = gates.reshape(na)
    order = jnp.argsort(eids, stable=True)
    sorted_e = eids[order]
    counts = jnp.zeros((NE,), jnp.int32).at[eids].add(1)
    padded = ((counts + BLK - 1) // BLK) * BLK
    pad_end = jnp.cumsum(padded)
    pad_off = pad_end - padded
    csum = jnp.cumsum(counts) - counts
    rank = jnp.arange(na, dtype=jnp.int32) - csum[sorted_e]
    dest = pad_off[sorted_e] + rank
    nrows = NB * BLK
    row_id = jnp.zeros((nrows,), jnp.int32).at[dest].set(
        (order // TOP2).astype(jnp.int32))
    gate_pad = jnp.zeros((nrows, 1), jnp.float32).at[dest, 0].set(gv[order])
    slots = jnp.zeros((na,), jnp.int32).at[order].set(dest).reshape(SEQ, TOP2)
    block_expert = jnp.minimum(
        jnp.searchsorted(pad_end, jnp.arange(NB, dtype=jnp.int32) * BLK,
                         side='right'), NE - 1).astype(jnp.int32)

    xg = jnp.take(t, row_id, axis=0)

    yp = pl.pallas_call(
        _moe_kernel,
        grid_spec=pltpu.PrefetchScalarGridSpec(
            num_scalar_prefetch=1,
            grid=(NB,),
            in_specs=[
                pl.BlockSpec((BLK, HID), lambda j, be: (j, 0)),
                pl.BlockSpec((1, HID, FFN), lambda j, be: (be[j], 0, 0)),
                pl.BlockSpec((1, 1, FFN), lambda j, be: (be[j], 0, 0)),
                pl.BlockSpec((1, FFN, HID), lambda j, be: (be[j], 0, 0)),
                pl.BlockSpec((1, 1, HID), lambda j, be: (be[j], 0, 0)),
                pl.BlockSpec((BLK, 1), lambda j, be: (j, 0)),
            ],
            out_specs=pl.BlockSpec((BLK, HID), lambda j, be: (j, 0)),
        ),
        out_shape=jax.ShapeDtypeStruct((nrows, HID), jnp.float32),
        compiler_params=pltpu.CompilerParams(
            dimension_semantics=("arbitrary",)),
    )(block_expert, xg, W1, b1.reshape(NE, 1, FFN), W2,
      b2.reshape(NE, 1, HID), gate_pad)

    out = h1 + jnp.take(yp, slots[:, 0], axis=0) + jnp.take(
        yp, slots[:, 1], axis=0)
    return out.reshape(1, SEQ, HID)


# speedup vs baseline: 1.2754x; 1.2754x over previous
import functools

import jax
import jax.numpy as jnp
from jax.experimental import pallas as pl
from jax.experimental.pallas import tpu as pltpu

HID = 768
HEADS = 12
DH = 64
NE = 8
TOP2 = 2
FFN = 768
SEQ = 2048
BLK = 128
NB = 40
ROWB = 256
QB = 512
NCH = 32


def _ln_qkv_kernel(x_ref, w_ref, b_ref, g_ref, be_ref, q_ref, k_ref, v_ref):
    x = x_ref[...]
    m = jnp.mean(x, axis=-1, keepdims=True)
    v = jnp.mean(jnp.square(x - m), axis=-1, keepdims=True)
    xn = (x - m) * jax.lax.rsqrt(v + 1e-5) * g_ref[...] + be_ref[...]
    for p, oref in enumerate((q_ref, k_ref, v_ref)):
        for h in range(HEADS):
            c = p * HEADS + h
            o = jax.lax.dot_general(xn, w_ref[c], (((1,), (1,)), ((), ())),
                                    preferred_element_type=jnp.float32)
            oref[h] = o + b_ref[c]


def _attn_kernel(q_ref, k_ref, v_ref, o_ref):
    q = q_ref[0]
    k = k_ref[0]
    v = v_ref[0]
    s = jax.lax.dot_general(q, k, (((1,), (1,)), ((), ())),
                            preferred_element_type=jnp.float32) * (DH ** -0.5)
    m = jnp.max(s, axis=-1, keepdims=True)
    p = jnp.exp(s - m)
    l = jnp.sum(p, axis=-1, keepdims=True)
    o = jnp.dot(p, v, preferred_element_type=jnp.float32) / l
    o_ref[0] = o


def _proj_ln2_route_kernel(a_ref, wo_ref, bo_ref, res_ref, g2_ref, b2_ref,
                           wr_ref, br_ref, h1_ref, t_ref, gates_ref, idx_ref):
    h1 = bo_ref[...] + res_ref[...]
    for h in range(HEADS):
        h1 = h1 + jnp.dot(a_ref[h], wo_ref[h],
                          preferred_element_type=jnp.float32)
    h1_ref[...] = h1
    m = jnp.mean(h1, axis=-1, keepdims=True)
    v = jnp.mean(jnp.square(h1 - m), axis=-1, keepdims=True)
    t = (h1 - m) * jax.lax.rsqrt(v + 1e-5) * g2_ref[...] + b2_ref[...]
    t_ref[...] = t
    logits = jnp.dot(t, wr_ref[...],
                     preferred_element_type=jnp.float32) + br_ref[...]
    lm = jnp.max(logits, axis=-1, keepdims=True)
    pe = jnp.exp(logits - lm)
    probs = pe / jnp.sum(pe, axis=-1, keepdims=True)
    i1 = jnp.argmax(probs, axis=-1)
    v1 = jnp.max(probs, axis=-1)
    cols = jax.lax.broadcasted_iota(jnp.int32, probs.shape, 1)
    masked = jnp.where(cols == i1[:, None], -jnp.inf, probs)
    i2 = jnp.argmax(masked, axis=-1)
    v2 = jnp.max(masked, axis=-1)
    tot = v1 + v2
    gates_ref[:, 0] = v1 / tot
    gates_ref[:, 1] = v2 / tot
    idx_ref[:, 0] = i1.astype(jnp.int32)
    idx_ref[:, 1] = i2.astype(jnp.int32)


def _route_rank_kernel(e_ref, dest_ref, be_ref):
    e = e_ref[...]
    rl = jax.lax.broadcasted_iota(jnp.int32, (128, 128), 0)
    cl = jax.lax.broadcasted_iota(jnp.int32, (128, 128), 1)
    T = (rl < cl).astype(jnp.float32)
    rc = jax.lax.broadcasted_iota(jnp.int32, (NCH, NCH), 0)
    cc = jax.lax.broadcasted_iota(jnp.int32, (NCH, NCH), 1)
    Tc = (cc < rc).astype(jnp.float32)
    dest = jnp.zeros((NCH, 128), jnp.float32)
    pad_off = jnp.zeros((1, 1), jnp.float32)
    jlane = jax.lax.broadcasted_iota(
        jnp.int32, (1, 128), 1).astype(jnp.float32) * BLK
    beacc = jnp.zeros((1, 128), jnp.float32)
    for ex in range(NE):
        A = (e == ex).astype(jnp.float32)
        P = jnp.dot(A, T, preferred_element_type=jnp.float32)
        tot = jnp.sum(A, axis=1, keepdims=True)
        C = jnp.dot(Tc, tot, preferred_element_type=jnp.float32)
        cnt = jnp.sum(tot)
        dest = dest + A * (P + C + pad_off)
        pad_off = pad_off + jnp.ceil(cnt / BLK) * BLK
        beacc = beacc + (jlane >= pad_off).astype(jnp.float32)
    dest_ref[...] = dest.astype(jnp.int32)
    be_ref[...] = jnp.minimum(beacc, NE - 1).astype(jnp.int32)


def _moe_kernel(be_ref, xg_ref, w1_ref, b1_ref, w2_ref, b2_ref, gate_ref,
                yp_ref):
    del be_ref
    xx = xg_ref[...]
    h = jnp.dot(xx, w1_ref[0], preferred_element_type=jnp.float32) + b1_ref[0]
    h = jax.nn.gelu(h)
    y = jnp.dot(h, w2_ref[0], preferred_element_type=jnp.float32) + b2_ref[0]
    yp_ref[...] = y * gate_ref[...]


def kernel(x, gamma1, beta1, Wqkv, bqkv, Wo, bo, gamma2, beta2, Wr, br,
           W1, b1, W2, b2):
    xf = x.reshape(SEQ, HID)

    q, k, v = pl.pallas_call(
        _ln_qkv_kernel,
        grid=(SEQ // ROWB,),
        in_specs=[
            pl.BlockSpec((ROWB, HID), lambda i: (i, 0)),
            pl.BlockSpec((3 * HEADS, DH, HID), lambda i: (0, 0, 0)),
            pl.BlockSpec((3 * HEADS, 1, DH), lambda i: (0, 0, 0)),
            pl.BlockSpec((1, HID), lambda i: (0, 0)),
            pl.BlockSpec((1, HID), lambda i: (0, 0)),
        ],
        out_specs=[
            pl.BlockSpec((HEADS, ROWB, DH), lambda i: (0, i, 0)),
            pl.BlockSpec((HEADS, ROWB, DH), lambda i: (0, i, 0)),
            pl.BlockSpec((HEADS, ROWB, DH), lambda i: (0, i, 0)),
        ],
        out_shape=[jax.ShapeDtypeStruct((HEADS, SEQ, DH), jnp.float32)] * 3,
        compiler_params=pltpu.CompilerParams(
            dimension_semantics=("parallel",)),
    )(xf, Wqkv.reshape(3 * HEADS, DH, HID), bqkv.reshape(3 * HEADS, 1, DH),
      gamma1.reshape(1, HID), beta1.reshape(1, HID))

    attn = pl.pallas_call(
        _attn_kernel,
        grid=(HEADS, SEQ // QB),
        in_specs=[
            pl.BlockSpec((1, QB, DH), lambda h, i: (h, i, 0)),
            pl.BlockSpec((1, SEQ, DH), lambda h, i: (h, 0, 0)),
            pl.BlockSpec((1, SEQ, DH), lambda h, i: (h, 0, 0)),
        ],
        out_specs=pl.BlockSpec((1, QB, DH), lambda h, i: (h, i, 0)),
        out_shape=jax.ShapeDtypeStruct((HEADS, SEQ, DH), jnp.float32),
        compiler_params=pltpu.CompilerParams(
            dimension_semantics=("parallel", "parallel")),
    )(q, k, v)

    Wo3 = jnp.transpose(Wo.reshape(HID, HEADS, DH), (1, 2, 0))
    h1, t, gates, idx = pl.pallas_call(
        _proj_ln2_route_kernel,
        grid=(SEQ // ROWB,),
        in_specs=[
            pl.BlockSpec((HEADS, ROWB, DH), lambda i: (0, i, 0)),
            pl.BlockSpec((HEADS, DH, HID), lambda i: (0, 0, 0)),
            pl.BlockSpec((1, HID), lambda i: (0, 0)),
            pl.BlockSpec((ROWB, HID), lambda i: (i, 0)),
            pl.BlockSpec((1, HID), lambda i: (0, 0)),
            pl.BlockSpec((1, HID), lambda i: (0, 0)),
            pl.BlockSpec((HID, NE), lambda i: (0, 0)),
            pl.BlockSpec((1, NE), lambda i: (0, 0)),
        ],
        out_specs=[
            pl.BlockSpec((ROWB, HID), lambda i: (i, 0)),
            pl.BlockSpec((ROWB, HID), lambda i: (i, 0)),
            pl.BlockSpec((ROWB, TOP2), lambda i: (i, 0)),
            pl.BlockSpec((ROWB, TOP2), lambda i: (i, 0)),
        ],
        out_shape=[
            jax.ShapeDtypeStruct((SEQ, HID), jnp.float32),
            jax.ShapeDtypeStruct((SEQ, HID), jnp.float32),
            jax.ShapeDtypeStruct((SEQ, TOP2), jnp.float32),
            jax.ShapeDtypeStruct((SEQ, TOP2), jnp.int32),
        ],
        compiler_params=pltpu.CompilerParams(
            dimension_semantics=("parallel",)),
    )(attn, Wo3, bo.reshape(1, HID), xf, gamma2.reshape(1, HID),
      beta2.reshape(1, HID), Wr, br.reshape(1, NE))

    na = SEQ * TOP2
    dest2, be2 = pl.pallas_call(
        _route_rank_kernel,
        in_specs=[pl.BlockSpec((NCH, 128), lambda: (0, 0))],
        out_specs=[
            pl.BlockSpec((NCH, 128), lambda: (0, 0)),
            pl.BlockSpec((1, 128), lambda: (0, 0)),
        ],
        out_shape=[
            jax.ShapeDtypeStruct((NCH, 128), jnp.int32),
            jax.ShapeDtypeStruct((1, 128), jnp.int32),
        ],
    )(idx.reshape(NCH, 128))

    destf = dest2.reshape(na)
    slots = destf.reshape(SEQ, TOP2)
    block_expert = be2[0, :NB]
    nrows = NB * BLK
    tok = jnp.arange(na, dtype=jnp.int32) // TOP2
    row_id = jnp.zeros((nrows,), jnp.int32).at[destf].set(tok)
    gate_pad = jnp.zeros((nrows,), jnp.float32).at[destf].set(
        gates.reshape(na)).reshape(nrows, 1)
    xg = jnp.take(t, row_id, axis=0)

    yp = pl.pallas_call(
        _moe_kernel,
        grid_spec=pltpu.PrefetchScalarGridSpec(
            num_scalar_prefetch=1,
            grid=(NB,),
            in_specs=[
                pl.BlockSpec((BLK, HID), lambda j, be: (j, 0)),
                pl.BlockSpec((1, HID, FFN), lambda j, be: (be[j], 0, 0)),
                pl.BlockSpec((1, 1, FFN), lambda j, be: (be[j], 0, 0)),
                pl.BlockSpec((1, FFN, HID), lambda j, be: (be[j], 0, 0)),
                pl.BlockSpec((1, 1, HID), lambda j, be: (be[j], 0, 0)),
                pl.BlockSpec((BLK, 1), lambda j, be: (j, 0)),
            ],
            out_specs=pl.BlockSpec((BLK, HID), lambda j, be: (j, 0)),
        ),
        out_shape=jax.ShapeDtypeStruct((nrows, HID), jnp.float32),
        compiler_params=pltpu.CompilerParams(
            dimension_semantics=("arbitrary",)),
    )(block_expert, xg, W1, b1.reshape(NE, 1, FFN), W2,
      b2.reshape(NE, 1, HID), gate_pad)

    out = h1 + jnp.take(yp, slots[:, 0], axis=0) + jnp.take(
        yp, slots[:, 1], axis=0)
    return out.reshape(1, SEQ, HID)


# speedup vs baseline: 1.3643x; 1.0697x over previous
import functools

import jax
import jax.numpy as jnp
from jax.experimental import pallas as pl
from jax.experimental.pallas import tpu as pltpu

HID = 768
HEADS = 12
DH = 64
NE = 8
TOP2 = 2
FFN = 768
SEQ = 2048
BLK = 128
NB = 40
ROWB = 256
QB = 512
NCH = 32


def _ln_qkv_kernel(x_ref, w_ref, b_ref, g_ref, be_ref, q_ref, k_ref, v_ref):
    x = x_ref[...]
    m = jnp.mean(x, axis=-1, keepdims=True)
    v = jnp.mean(jnp.square(x - m), axis=-1, keepdims=True)
    xn = (x - m) * jax.lax.rsqrt(v + 1e-5) * g_ref[...] + be_ref[...]
    o = jnp.dot(xn, w_ref[...],
                preferred_element_type=jnp.float32) + b_ref[...]
    ob = o.astype(jnp.bfloat16)
    for p, oref in enumerate((q_ref, k_ref, v_ref)):
        for h in range(HEADS):
            c = p * HEADS + h
            oref[h] = ob[:, c * DH:(c + 1) * DH]


def _attn_kernel(q_ref, k_ref, v_ref, o_ref):
    q = q_ref[0]
    k = k_ref[0]
    v = v_ref[0]
    s = jax.lax.dot_general(q, k, (((1,), (1,)), ((), ())),
                            preferred_element_type=jnp.float32) * (DH ** -0.5)
    m = jnp.max(s, axis=-1, keepdims=True)
    p = jnp.exp(s - m)
    l = jnp.sum(p, axis=-1, keepdims=True)
    o = jnp.dot(p.astype(jnp.bfloat16), v,
                preferred_element_type=jnp.float32)
    o_ref[0] = (o * (1.0 / l)).astype(jnp.bfloat16)


def _proj_ln2_route_kernel(a_ref, wo_ref, bo_ref, res_ref, g2_ref, b2_ref,
                           wr_ref, br_ref, h1_ref, t_ref, gates_ref, idx_ref,
                           acc_ref):
    for h in range(HEADS):
        acc_ref[:, h * DH:(h + 1) * DH] = a_ref[h]
    h1 = jnp.dot(acc_ref[...], wo_ref[...],
                 preferred_element_type=jnp.float32)
    h1 = h1 + bo_ref[...] + res_ref[...]
    h1_ref[...] = h1
    m = jnp.mean(h1, axis=-1, keepdims=True)
    v = jnp.mean(jnp.square(h1 - m), axis=-1, keepdims=True)
    t = (h1 - m) * jax.lax.rsqrt(v + 1e-5) * g2_ref[...] + b2_ref[...]
    t_ref[...] = t
    logits = jnp.dot(t, wr_ref[...],
                     preferred_element_type=jnp.float32) + br_ref[...]
    lm = jnp.max(logits, axis=-1, keepdims=True)
    pe = jnp.exp(logits - lm)
    probs = pe / jnp.sum(pe, axis=-1, keepdims=True)
    i1 = jnp.argmax(probs, axis=-1)
    v1 = jnp.max(probs, axis=-1)
    cols = jax.lax.broadcasted_iota(jnp.int32, probs.shape, 1)
    masked = jnp.where(cols == i1[:, None], -jnp.inf, probs)
    i2 = jnp.argmax(masked, axis=-1)
    v2 = jnp.max(masked, axis=-1)
    tot = v1 + v2
    gates_ref[:, 0] = v1 / tot
    gates_ref[:, 1] = v2 / tot
    idx_ref[:, 0] = i1.astype(jnp.int32)
    idx_ref[:, 1] = i2.astype(jnp.int32)


def _route_rank_kernel(e_ref, dest_ref, be_ref):
    e = e_ref[...]
    rl = jax.lax.broadcasted_iota(jnp.int32, (128, 128), 0)
    cl = jax.lax.broadcasted_iota(jnp.int32, (128, 128), 1)
    T = (rl < cl).astype(jnp.float32)
    rc = jax.lax.broadcasted_iota(jnp.int32, (NCH, NCH), 0)
    cc = jax.lax.broadcasted_iota(jnp.int32, (NCH, NCH), 1)
    Tc = (cc < rc).astype(jnp.float32)
    dest = jnp.zeros((NCH, 128), jnp.float32)
    pad_off = jnp.zeros((1, 1), jnp.float32)
    jlane = jax.lax.broadcasted_iota(
        jnp.int32, (1, 128), 1).astype(jnp.float32) * BLK
    beacc = jnp.zeros((1, 128), jnp.float32)
    for ex in range(NE):
        A = (e == ex).astype(jnp.float32)
        P = jnp.dot(A, T, preferred_element_type=jnp.float32)
        tot = jnp.sum(A, axis=1, keepdims=True)
        C = jnp.dot(Tc, tot, preferred_element_type=jnp.float32)
        cnt = jnp.sum(tot)
        dest = dest + A * (P + C + pad_off)
        pad_off = pad_off + jnp.ceil(cnt / BLK) * BLK
        beacc = beacc + (jlane >= pad_off).astype(jnp.float32)
    dest_ref[...] = dest.astype(jnp.int32)
    be_ref[...] = jnp.minimum(beacc, NE - 1).astype(jnp.int32)


def _moe_kernel(be_ref, xg_ref, w1_ref, b1_ref, w2_ref, b2_ref, gate_ref,
                yp_ref):
    del be_ref
    xx = xg_ref[...]
    h = jnp.dot(xx, w1_ref[0], preferred_element_type=jnp.float32) + b1_ref[0]
    h = jax.nn.gelu(h)
    y = jnp.dot(h, w2_ref[0], preferred_element_type=jnp.float32) + b2_ref[0]
    yp_ref[...] = y * gate_ref[...]


def kernel(x, gamma1, beta1, Wqkv, bqkv, Wo, bo, gamma2, beta2, Wr, br,
           W1, b1, W2, b2):
    xf = x.reshape(SEQ, HID)

    q, k, v = pl.pallas_call(
        _ln_qkv_kernel,
        grid=(SEQ // ROWB,),
        in_specs=[
            pl.BlockSpec((ROWB, HID), lambda i: (i, 0)),
            pl.BlockSpec((HID, 3 * HID), lambda i: (0, 0)),
            pl.BlockSpec((1, 3 * HID), lambda i: (0, 0)),
            pl.BlockSpec((1, HID), lambda i: (0, 0)),
            pl.BlockSpec((1, HID), lambda i: (0, 0)),
        ],
        out_specs=[
            pl.BlockSpec((HEADS, ROWB, DH), lambda i: (0, i, 0)),
            pl.BlockSpec((HEADS, ROWB, DH), lambda i: (0, i, 0)),
            pl.BlockSpec((HEADS, ROWB, DH), lambda i: (0, i, 0)),
        ],
        out_shape=[jax.ShapeDtypeStruct((HEADS, SEQ, DH), jnp.bfloat16)] * 3,
        compiler_params=pltpu.CompilerParams(
            dimension_semantics=("parallel",)),
    )(xf, Wqkv.T, bqkv.reshape(1, 3 * HID),
      gamma1.reshape(1, HID), beta1.reshape(1, HID))

    attn = pl.pallas_call(
        _attn_kernel,
        grid=(HEADS, SEQ // QB),
        in_specs=[
            pl.BlockSpec((1, QB, DH), lambda h, i: (h, i, 0)),
            pl.BlockSpec((1, SEQ, DH), lambda h, i: (h, 0, 0)),
            pl.BlockSpec((1, SEQ, DH), lambda h, i: (h, 0, 0)),
        ],
        out_specs=pl.BlockSpec((1, QB, DH), lambda h, i: (h, i, 0)),
        out_shape=jax.ShapeDtypeStruct((HEADS, SEQ, DH), jnp.bfloat16),
        compiler_params=pltpu.CompilerParams(
            dimension_semantics=("parallel", "parallel")),
    )(q, k, v)

    WoT = Wo.T.astype(jnp.bfloat16)
    h1, t, gates, idx = pl.pallas_call(
        _proj_ln2_route_kernel,
        grid=(SEQ // ROWB,),
        in_specs=[
            pl.BlockSpec((HEADS, ROWB, DH), lambda i: (0, i, 0)),
            pl.BlockSpec((HID, HID), lambda i: (0, 0)),
            pl.BlockSpec((1, HID), lambda i: (0, 0)),
            pl.BlockSpec((ROWB, HID), lambda i: (i, 0)),
            pl.BlockSpec((1, HID), lambda i: (0, 0)),
            pl.BlockSpec((1, HID), lambda i: (0, 0)),
            pl.BlockSpec((HID, NE), lambda i: (0, 0)),
            pl.BlockSpec((1, NE), lambda i: (0, 0)),
        ],
        out_specs=[
            pl.BlockSpec((ROWB, HID), lambda i: (i, 0)),
            pl.BlockSpec((ROWB, HID), lambda i: (i, 0)),
            pl.BlockSpec((ROWB, TOP2), lambda i: (i, 0)),
            pl.BlockSpec((ROWB, TOP2), lambda i: (i, 0)),
        ],
        out_shape=[
            jax.ShapeDtypeStruct((SEQ, HID), jnp.float32),
            jax.ShapeDtypeStruct((SEQ, HID), jnp.float32),
            jax.ShapeDtypeStruct((SEQ, TOP2), jnp.float32),
            jax.ShapeDtypeStruct((SEQ, TOP2), jnp.int32),
        ],
        scratch_shapes=[pltpu.VMEM((ROWB, HID), jnp.bfloat16)],
        compiler_params=pltpu.CompilerParams(
            dimension_semantics=("parallel",)),
    )(attn, WoT, bo.reshape(1, HID), xf, gamma2.reshape(1, HID),
      beta2.reshape(1, HID), Wr, br.reshape(1, NE))

    na = SEQ * TOP2
    dest2, be2 = pl.pallas_call(
        _route_rank_kernel,
        in_specs=[pl.BlockSpec((NCH, 128), lambda: (0, 0))],
        out_specs=[
            pl.BlockSpec((NCH, 128), lambda: (0, 0)),
            pl.BlockSpec((1, 128), lambda: (0, 0)),
        ],
        out_shape=[
            jax.ShapeDtypeStruct((NCH, 128), jnp.int32),
            jax.ShapeDtypeStruct((1, 128), jnp.int32),
        ],
    )(idx.reshape(NCH, 128))

    destf = dest2.reshape(na)
    slots = destf.reshape(SEQ, TOP2)
    block_expert = be2[0, :NB]
    nrows = NB * BLK
    tok = jnp.arange(na, dtype=jnp.int32) // TOP2
    row_id = jnp.zeros((nrows,), jnp.int32).at[destf].set(tok)
    gate_pad = jnp.zeros((nrows,), jnp.float32).at[destf].set(
        gates.reshape(na)).reshape(nrows, 1)
    xg = jnp.take(t, row_id, axis=0)

    yp = pl.pallas_call(
        _moe_kernel,
        grid_spec=pltpu.PrefetchScalarGridSpec(
            num_scalar_prefetch=1,
            grid=(NB,),
            in_specs=[
                pl.BlockSpec((BLK, HID), lambda j, be: (j, 0)),
                pl.BlockSpec((1, HID, FFN), lambda j, be: (be[j], 0, 0)),
                pl.BlockSpec((1, 1, FFN), lambda j, be: (be[j], 0, 0)),
                pl.BlockSpec((1, FFN, HID), lambda j, be: (be[j], 0, 0)),
                pl.BlockSpec((1, 1, HID), lambda j, be: (be[j], 0, 0)),
                pl.BlockSpec((BLK, 1), lambda j, be: (j, 0)),
            ],
            out_specs=pl.BlockSpec((BLK, HID), lambda j, be: (j, 0)),
        ),
        out_shape=jax.ShapeDtypeStruct((nrows, HID), jnp.float32),
        compiler_params=pltpu.CompilerParams(
            dimension_semantics=("arbitrary",)),
    )(block_expert, xg, W1, b1.reshape(NE, 1, FFN), W2,
      b2.reshape(NE, 1, HID), gate_pad)

    out = h1 + jnp.take(yp, slots[:, 0], axis=0) + jnp.take(
        yp, slots[:, 1], axis=0)
    return out.reshape(1, SEQ, HID)


# speedup vs baseline: 1.8911x; 1.3861x over previous
import functools

import jax
import jax.numpy as jnp
from jax.experimental import pallas as pl
from jax.experimental.pallas import tpu as pltpu

HID = 768
HEADS = 12
DH = 64
NE = 8
TOP2 = 2
FFN = 768
SEQ = 2048
ROWB = 256
QB = 512


def _ln_qkv_kernel(x_ref, w_ref, b_ref, g_ref, be_ref, q_ref, k_ref, v_ref):
    x = x_ref[...]
    m = jnp.mean(x, axis=-1, keepdims=True)
    v = jnp.mean(jnp.square(x - m), axis=-1, keepdims=True)
    xn = (x - m) * jax.lax.rsqrt(v + 1e-5) * g_ref[...] + be_ref[...]
    o = jnp.dot(xn, w_ref[...],
                preferred_element_type=jnp.float32) + b_ref[...]
    ob = o.astype(jnp.bfloat16)
    for p, oref in enumerate((q_ref, k_ref, v_ref)):
        for h in range(HEADS):
            c = p * HEADS + h
            oref[h] = ob[:, c * DH:(c + 1) * DH]


def _attn_kernel(q_ref, k_ref, v_ref, o_ref):
    q = q_ref[0]
    k = k_ref[0]
    v = v_ref[0]
    s = jax.lax.dot_general(q, k, (((1,), (1,)), ((), ())),
                            preferred_element_type=jnp.float32) * (DH ** -0.5)
    sb = s.astype(jnp.bfloat16)
    m = jnp.max(sb, axis=-1, keepdims=True)
    pb = jnp.exp((sb - m).astype(jnp.float32)).astype(jnp.bfloat16)
    l = jnp.sum(pb, axis=-1, keepdims=True, dtype=jnp.float32)
    o = jnp.dot(pb, v, preferred_element_type=jnp.float32)
    o_ref[0] = (o * (1.0 / l)).astype(jnp.bfloat16)


def _block_moe_kernel(a_ref, wo_ref, bo_ref, res_ref, g2_ref, b2ln_ref,
                      wr_ref, br_ref, w1_ref, b1_ref, w2_ref, b2_ref,
                      o_ref, acc_ref):
    for h in range(HEADS):
        acc_ref[:, h * DH:(h + 1) * DH] = a_ref[h]
    h1 = jnp.dot(acc_ref[...], wo_ref[...],
                 preferred_element_type=jnp.float32)
    h1 = h1 + bo_ref[...] + res_ref[...]
    m = jnp.mean(h1, axis=-1, keepdims=True)
    va = jnp.mean(jnp.square(h1 - m), axis=-1, keepdims=True)
    t = (h1 - m) * jax.lax.rsqrt(va + 1e-5) * g2_ref[...] + b2ln_ref[...]
    logits = jnp.dot(t, wr_ref[...],
                     preferred_element_type=jnp.float32) + br_ref[...]
    lm = jnp.max(logits, axis=-1, keepdims=True)
    pe = jnp.exp(logits - lm)
    probs = pe / jnp.sum(pe, axis=-1, keepdims=True)
    v1 = jnp.max(probs, axis=-1, keepdims=True)
    cols = jax.lax.broadcasted_iota(jnp.int32, probs.shape, 1)
    i1 = jnp.argmax(probs, axis=-1)
    masked = jnp.where(cols == i1[:, None], -jnp.inf, probs)
    v2 = jnp.max(masked, axis=-1, keepdims=True)
    i2 = jnp.argmax(masked, axis=-1)
    tot = v1 + v2
    gates = jnp.where(cols == i1[:, None], v1 / tot,
                      jnp.where(cols == i2[:, None], v2 / tot, 0.0))
    out = h1
    for e in range(NE):
        hm = jnp.dot(t, w1_ref[e],
                     preferred_element_type=jnp.float32) + b1_ref[e]
        hm = jax.nn.gelu(hm)
        y = jnp.dot(hm, w2_ref[e],
                    preferred_element_type=jnp.float32) + b2_ref[e]
        out = out + y * gates[:, e:e + 1]
    o_ref[...] = out


def kernel(x, gamma1, beta1, Wqkv, bqkv, Wo, bo, gamma2, beta2, Wr, br,
           W1, b1, W2, b2):
    xf = x.reshape(SEQ, HID)

    q, k, v = pl.pallas_call(
        _ln_qkv_kernel,
        grid=(SEQ // ROWB,),
        in_specs=[
            pl.BlockSpec((ROWB, HID), lambda i: (i, 0)),
            pl.BlockSpec((HID, 3 * HID), lambda i: (0, 0)),
            pl.BlockSpec((1, 3 * HID), lambda i: (0, 0)),
            pl.BlockSpec((1, HID), lambda i: (0, 0)),
            pl.BlockSpec((1, HID), lambda i: (0, 0)),
        ],
        out_specs=[
            pl.BlockSpec((HEADS, ROWB, DH), lambda i: (0, i, 0)),
            pl.BlockSpec((HEADS, ROWB, DH), lambda i: (0, i, 0)),
            pl.BlockSpec((HEADS, ROWB, DH), lambda i: (0, i, 0)),
        ],
        out_shape=[jax.ShapeDtypeStruct((HEADS, SEQ, DH), jnp.bfloat16)] * 3,
        compiler_params=pltpu.CompilerParams(
            dimension_semantics=("parallel",)),
    )(xf, Wqkv.T, bqkv.reshape(1, 3 * HID),
      gamma1.reshape(1, HID), beta1.reshape(1, HID))

    attn = pl.pallas_call(
        _attn_kernel,
        grid=(HEADS, SEQ // QB),
        in_specs=[
            pl.BlockSpec((1, QB, DH), lambda h, i: (h, i, 0)),
            pl.BlockSpec((1, SEQ, DH), lambda h, i: (h, 0, 0)),
            pl.BlockSpec((1, SEQ, DH), lambda h, i: (h, 0, 0)),
        ],
        out_specs=pl.BlockSpec((1, QB, DH), lambda h, i: (h, i, 0)),
        out_shape=jax.ShapeDtypeStruct((HEADS, SEQ, DH), jnp.bfloat16),
        compiler_params=pltpu.CompilerParams(
            dimension_semantics=("parallel", "parallel")),
    )(q, k, v)

    WoT = Wo.T.astype(jnp.bfloat16)
    out = pl.pallas_call(
        _block_moe_kernel,
        grid=(SEQ // ROWB,),
        in_specs=[
            pl.BlockSpec((HEADS, ROWB, DH), lambda i: (0, i, 0)),
            pl.BlockSpec((HID, HID), lambda i: (0, 0)),
            pl.BlockSpec((1, HID), lambda i: (0, 0)),
            pl.BlockSpec((ROWB, HID), lambda i: (i, 0)),
            pl.BlockSpec((1, HID), lambda i: (0, 0)),
            pl.BlockSpec((1, HID), lambda i: (0, 0)),
            pl.BlockSpec((HID, NE), lambda i: (0, 0)),
            pl.BlockSpec((1, NE), lambda i: (0, 0)),
            pl.BlockSpec((NE, HID, FFN), lambda i: (0, 0, 0)),
            pl.BlockSpec((NE, 1, FFN), lambda i: (0, 0, 0)),
            pl.BlockSpec((NE, FFN, HID), lambda i: (0, 0, 0)),
            pl.BlockSpec((NE, 1, HID), lambda i: (0, 0, 0)),
        ],
        out_specs=pl.BlockSpec((ROWB, HID), lambda i: (i, 0)),
        out_shape=jax.ShapeDtypeStruct((SEQ, HID), jnp.float32),
        scratch_shapes=[pltpu.VMEM((ROWB, HID), jnp.bfloat16)],
        compiler_params=pltpu.CompilerParams(
            dimension_semantics=("arbitrary",)),
    )(attn, WoT, bo.reshape(1, HID), xf, gamma2.reshape(1, HID),
      beta2.reshape(1, HID), Wr, br.reshape(1, NE),
      W1, b1.reshape(NE, 1, FFN), W2, b2.reshape(NE, 1, HID))

    return out.reshape(1, SEQ, HID)


# speedup vs baseline: 1.9868x; 1.0506x over previous
import functools

import jax
import jax.numpy as jnp
from jax.experimental import pallas as pl
from jax.experimental.pallas import tpu as pltpu

HID = 768
HEADS = 12
DH = 64
NE = 8
TOP2 = 2
FFN = 768
SEQ = 2048
ROWB = 256
QB = 1024
WCH = HEADS * (SEQ // QB)
WROW = NE * HID // WCH


def _ln_qkv_kernel(x_ref, w_ref, b_ref, g_ref, be_ref, q_ref, k_ref, v_ref):
    x = x_ref[...]
    m = jnp.mean(x, axis=-1, keepdims=True)
    v = jnp.mean(jnp.square(x - m), axis=-1, keepdims=True)
    xn = (x - m) * jax.lax.rsqrt(v + 1e-5) * g_ref[...] + be_ref[...]
    o = jnp.dot(xn, w_ref[...],
                preferred_element_type=jnp.float32) + b_ref[...]
    ob = o.astype(jnp.bfloat16)
    for p, oref in enumerate((q_ref, k_ref, v_ref)):
        for h in range(HEADS):
            c = p * HEADS + h
            oref[h] = ob[:, c * DH:(c + 1) * DH]


def _attn_kernel(q_ref, k_ref, v_ref, w1_ref, w2_ref, o_ref, w1b_ref,
                 w2b_ref):
    w1b_ref[...] = w1_ref[...].astype(jnp.bfloat16)
    w2b_ref[...] = w2_ref[...].astype(jnp.bfloat16)
    q = q_ref[0]
    k = k_ref[0]
    v = v_ref[0]
    s = jax.lax.dot_general(q, k, (((1,), (1,)), ((), ())),
                            preferred_element_type=jnp.float32) * (DH ** -0.5)
    sb = s.astype(jnp.bfloat16)
    m = jnp.max(sb, axis=-1, keepdims=True)
    pb = jnp.exp((sb - m).astype(jnp.float32)).astype(jnp.bfloat16)
    l = jnp.sum(pb, axis=-1, keepdims=True, dtype=jnp.float32)
    o = jnp.dot(pb, v, preferred_element_type=jnp.float32)
    o_ref[0] = (o * (1.0 / l)).astype(jnp.bfloat16)


def _block_moe_kernel(a_ref, wo_ref, bo_ref, res_ref, g2_ref, b2ln_ref,
                      wr_ref, br_ref, w1_ref, b1_ref, w2_ref, b2_ref,
                      o_ref, acc_ref):
    for h in range(HEADS):
        acc_ref[:, h * DH:(h + 1) * DH] = a_ref[h]
    h1 = jnp.dot(acc_ref[...], wo_ref[...],
                 preferred_element_type=jnp.float32)
    h1 = h1 + bo_ref[...] + res_ref[...]
    m = jnp.mean(h1, axis=-1, keepdims=True)
    va = jnp.mean(jnp.square(h1 - m), axis=-1, keepdims=True)
    t = (h1 - m) * jax.lax.rsqrt(va + 1e-5) * g2_ref[...] + b2ln_ref[...]
    logits = jnp.dot(t, wr_ref[...],
                     preferred_element_type=jnp.float32) + br_ref[...]
    lm = jnp.max(logits, axis=-1, keepdims=True)
    pe = jnp.exp(logits - lm)
    probs = pe / jnp.sum(pe, axis=-1, keepdims=True)
    v1 = jnp.max(probs, axis=-1, keepdims=True)
    cols = jax.lax.broadcasted_iota(jnp.int32, probs.shape, 1)
    i1 = jnp.argmax(probs, axis=-1)
    masked = jnp.where(cols == i1[:, None], -jnp.inf, probs)
    v2 = jnp.max(masked, axis=-1, keepdims=True)
    i2 = jnp.argmax(masked, axis=-1)
    tot = v1 + v2
    gates = jnp.where(cols == i1[:, None], v1 / tot,
                      jnp.where(cols == i2[:, None], v2 / tot, 0.0))
    out = h1
    tb = t.astype(jnp.bfloat16)
    for e in range(NE):
        hm = jnp.dot(tb, w1_ref[e],
                     preferred_element_type=jnp.float32) + b1_ref[e]
        hm = jax.nn.gelu(hm)
        y = jnp.dot(hm.astype(jnp.bfloat16), w2_ref[e],
                    preferred_element_type=jnp.float32) + b2_ref[e]
        out = out + y * gates[:, e:e + 1]
    o_ref[...] = out


def kernel(x, gamma1, beta1, Wqkv, bqkv, Wo, bo, gamma2, beta2, Wr, br,
           W1, b1, W2, b2):
    xf = x.reshape(SEQ, HID)

    q, k, v = pl.pallas_call(
        _ln_qkv_kernel,
        grid=(SEQ // ROWB,),
        in_specs=[
            pl.BlockSpec((ROWB, HID), lambda i: (i, 0)),
            pl.BlockSpec((HID, 3 * HID), lambda i: (0, 0)),
            pl.BlockSpec((1, 3 * HID), lambda i: (0, 0)),
            pl.BlockSpec((1, HID), lambda i: (0, 0)),
            pl.BlockSpec((1, HID), lambda i: (0, 0)),
        ],
        out_specs=[
            pl.BlockSpec((HEADS, ROWB, DH), lambda i: (0, i, 0)),
            pl.BlockSpec((HEADS, ROWB, DH), lambda i: (0, i, 0)),
            pl.BlockSpec((HEADS, ROWB, DH), lambda i: (0, i, 0)),
        ],
        out_shape=[jax.ShapeDtypeStruct((HEADS, SEQ, DH), jnp.bfloat16)] * 3,
        compiler_params=pltpu.CompilerParams(
            dimension_semantics=("parallel",)),
    )(xf, Wqkv.T, bqkv.reshape(1, 3 * HID),
      gamma1.reshape(1, HID), beta1.reshape(1, HID))

    nq = SEQ // QB
    attn, W1b, W2b = pl.pallas_call(
        _attn_kernel,
        grid=(HEADS, nq),
        in_specs=[
            pl.BlockSpec((1, QB, DH), lambda h, i: (h, i, 0)),
            pl.BlockSpec((1, SEQ, DH), lambda h, i: (h, 0, 0)),
            pl.BlockSpec((1, SEQ, DH), lambda h, i: (h, 0, 0)),
            pl.BlockSpec((1, WROW, FFN), lambda h, i: (h * nq + i, 0, 0)),
            pl.BlockSpec((1, WROW, HID), lambda h, i: (h * nq + i, 0, 0)),
        ],
        out_specs=[
            pl.BlockSpec((1, QB, DH), lambda h, i: (h, i, 0)),
            pl.BlockSpec((1, WROW, FFN), lambda h, i: (h * nq + i, 0, 0)),
            pl.BlockSpec((1, WROW, HID), lambda h, i: (h * nq + i, 0, 0)),
        ],
        out_shape=[
            jax.ShapeDtypeStruct((HEADS, SEQ, DH), jnp.bfloat16),
            jax.ShapeDtypeStruct((WCH, WROW, FFN), jnp.bfloat16),
            jax.ShapeDtypeStruct((WCH, WROW, HID), jnp.bfloat16),
        ],
        compiler_params=pltpu.CompilerParams(
            dimension_semantics=("parallel", "parallel")),
    )(q, k, v, W1.reshape(WCH, WROW, FFN), W2.reshape(WCH, WROW, HID))
    W1b = W1b.reshape(NE, HID, FFN)
    W2b = W2b.reshape(NE, FFN, HID)

    WoT = Wo.T.astype(jnp.bfloat16)
    out = pl.pallas_call(
        _block_moe_kernel,
        grid=(SEQ // ROWB,),
        in_specs=[
            pl.BlockSpec((HEADS, ROWB, DH), lambda i: (0, i, 0)),
            pl.BlockSpec((HID, HID), lambda i: (0, 0)),
            pl.BlockSpec((1, HID), lambda i: (0, 0)),
            pl.BlockSpec((ROWB, HID), lambda i: (i, 0)),
            pl.BlockSpec((1, HID), lambda i: (0, 0)),
            pl.BlockSpec((1, HID), lambda i: (0, 0)),
            pl.BlockSpec((HID, NE), lambda i: (0, 0)),
            pl.BlockSpec((1, NE), lambda i: (0, 0)),
            pl.BlockSpec((NE, HID, FFN), lambda i: (0, 0, 0)),
            pl.BlockSpec((NE, 1, FFN), lambda i: (0, 0, 0)),
            pl.BlockSpec((NE, FFN, HID), lambda i: (0, 0, 0)),
            pl.BlockSpec((NE, 1, HID), lambda i: (0, 0, 0)),
        ],
        out_specs=pl.BlockSpec((ROWB, HID), lambda i: (i, 0)),
        out_shape=jax.ShapeDtypeStruct((SEQ, HID), jnp.float32),
        scratch_shapes=[pltpu.VMEM((ROWB, HID), jnp.bfloat16)],
        compiler_params=pltpu.CompilerParams(
            dimension_semantics=("arbitrary",)),
    )(attn, WoT, bo.reshape(1, HID), xf, gamma2.reshape(1, HID),
      beta2.reshape(1, HID), Wr, br.reshape(1, NE),
      W1b, b1.reshape(NE, 1, FFN), W2b, b2.reshape(NE, 1, HID))

    return out.reshape(1, SEQ, HID)


# speedup vs baseline: 2.3806x; 1.1982x over previous
import functools

import jax
import jax.numpy as jnp
from jax.experimental import pallas as pl
from jax.experimental.pallas import tpu as pltpu

HID = 768
HEADS = 12
DH = 64
NE = 8
TOP2 = 2
FFN = 768
SEQ = 2048
ROWB = 256
QB = 1024
HP = 2
WCH = (HEADS // HP) * (SEQ // QB)
WROW = NE * HID // WCH
QSCALE = (DH ** -0.5) * 1.4426950408889634


def _ln_qkv_kernel(x_ref, w_ref, b_ref, g_ref, be_ref, q_ref, k_ref, v_ref):
    x = x_ref[...]
    m = jnp.mean(x, axis=-1, keepdims=True)
    v = jnp.mean(jnp.square(x - m), axis=-1, keepdims=True)
    xn = (x - m) * jax.lax.rsqrt(v + 1e-5) * g_ref[...] + be_ref[...]
    o = jnp.dot(xn, w_ref[...],
                preferred_element_type=jnp.float32) + b_ref[...]
    for p, oref in enumerate((q_ref, k_ref, v_ref)):
        for h in range(HEADS):
            c = p * HEADS + h
            sl = o[:, c * DH:(c + 1) * DH]
            if p == 0:
                sl = sl * QSCALE
            oref[h] = sl.astype(jnp.bfloat16)


def _attn_kernel(q_ref, k_ref, v_ref, w1_ref, w2_ref, o_ref, w1b_ref,
                 w2b_ref):
    w1b_ref[...] = w1_ref[...].astype(jnp.bfloat16)
    w2b_ref[...] = w2_ref[...].astype(jnp.bfloat16)
    for j in range(HP):
        q = q_ref[j]
        k = k_ref[j]
        v = v_ref[j]
        s = jax.lax.dot_general(q, k, (((1,), (1,)), ((), ())),
                                preferred_element_type=jnp.float32)
        sb = s.astype(jnp.bfloat16)
        m = jnp.max(sb, axis=-1, keepdims=True)
        pb = jnp.exp2((sb - m).astype(jnp.float32)).astype(jnp.bfloat16)
        l = jnp.sum(pb, axis=-1, keepdims=True, dtype=jnp.float32)
        o = jnp.dot(pb, v, preferred_element_type=jnp.float32)
        o_ref[j] = (o * (1.0 / l)).astype(jnp.bfloat16)


def _block_moe_kernel(a_ref, wo_ref, bo_ref, res_ref, g2_ref, b2ln_ref,
                      wr_ref, br_ref, w1_ref, b1_ref, w2_ref, b2_ref,
                      o_ref, acc_ref):
    for h in range(HEADS):
        acc_ref[:, h * DH:(h + 1) * DH] = a_ref[h]
    h1 = jnp.dot(acc_ref[...], wo_ref[...],
                 preferred_element_type=jnp.float32)
    h1 = h1 + bo_ref[...] + res_ref[...]
    m = jnp.mean(h1, axis=-1, keepdims=True)
    va = jnp.mean(jnp.square(h1 - m), axis=-1, keepdims=True)
    t = (h1 - m) * jax.lax.rsqrt(va + 1e-5) * g2_ref[...] + b2ln_ref[...]
    logits = jnp.dot(t, wr_ref[...],
                     preferred_element_type=jnp.float32) + br_ref[...]
    lm = jnp.max(logits, axis=-1, keepdims=True)
    pe = jnp.exp(logits - lm)
    probs = pe / jnp.sum(pe, axis=-1, keepdims=True)
    v1 = jnp.max(probs, axis=-1, keepdims=True)
    cols = jax.lax.broadcasted_iota(jnp.int32, probs.shape, 1)
    i1 = jnp.argmax(probs, axis=-1)
    masked = jnp.where(cols == i1[:, None], -jnp.inf, probs)
    v2 = jnp.max(masked, axis=-1, keepdims=True)
    i2 = jnp.argmax(masked, axis=-1)
    tot = v1 + v2
    gates = jnp.where(cols == i1[:, None], v1 / tot,
                      jnp.where(cols == i2[:, None], v2 / tot, 0.0))
    out = h1
    tb = t.astype(jnp.bfloat16)
    for e in range(NE):
        hm = jnp.dot(tb, w1_ref[e],
                     preferred_element_type=jnp.float32) + b1_ref[e]
        hm = jax.nn.gelu(hm)
        y = jnp.dot(hm.astype(jnp.bfloat16), w2_ref[e],
                    preferred_element_type=jnp.float32) + b2_ref[e]
        out = out + y * gates[:, e:e + 1]
    o_ref[...] = out


def kernel(x, gamma1, beta1, Wqkv, bqkv, Wo, bo, gamma2, beta2, Wr, br,
           W1, b1, W2, b2):
    xf = x.reshape(SEQ, HID)

    q, k, v = pl.pallas_call(
        _ln_qkv_kernel,
        grid=(SEQ // ROWB,),
        in_specs=[
            pl.BlockSpec((ROWB, HID), lambda i: (i, 0)),
            pl.BlockSpec((HID, 3 * HID), lambda i: (0, 0)),
            pl.BlockSpec((1, 3 * HID), lambda i: (0, 0)),
            pl.BlockSpec((1, HID), lambda i: (0, 0)),
            pl.BlockSpec((1, HID), lambda i: (0, 0)),
        ],
        out_specs=[
            pl.BlockSpec((HEADS, ROWB, DH), lambda i: (0, i, 0)),
            pl.BlockSpec((HEADS, ROWB, DH), lambda i: (0, i, 0)),
            pl.BlockSpec((HEADS, ROWB, DH), lambda i: (0, i, 0)),
        ],
        out_shape=[jax.ShapeDtypeStruct((HEADS, SEQ, DH), jnp.bfloat16)] * 3,
        compiler_params=pltpu.CompilerParams(
            dimension_semantics=("parallel",)),
    )(xf, Wqkv.T, bqkv.reshape(1, 3 * HID),
      gamma1.reshape(1, HID), beta1.reshape(1, HID))

    nq = SEQ // QB
    attn, W1b, W2b = pl.pallas_call(
        _attn_kernel,
        grid=(HEADS // HP, nq),
        in_specs=[
            pl.BlockSpec((HP, QB, DH), lambda h, i: (h, i, 0)),
            pl.BlockSpec((HP, SEQ, DH), lambda h, i: (h, 0, 0)),
            pl.BlockSpec((HP, SEQ, DH), lambda h, i: (h, 0, 0)),
            pl.BlockSpec((1, WROW, FFN), lambda h, i: (h * nq + i, 0, 0)),
            pl.BlockSpec((1, WROW, HID), lambda h, i: (h * nq + i, 0, 0)),
        ],
        out_specs=[
            pl.BlockSpec((HP, QB, DH), lambda h, i: (h, i, 0)),
            pl.BlockSpec((1, WROW, FFN), lambda h, i: (h * nq + i, 0, 0)),
            pl.BlockSpec((1, WROW, HID), lambda h, i: (h * nq + i, 0, 0)),
        ],
        out_shape=[
            jax.ShapeDtypeStruct((HEADS, SEQ, DH), jnp.bfloat16),
            jax.ShapeDtypeStruct((WCH, WROW, FFN), jnp.bfloat16),
            jax.ShapeDtypeStruct((WCH, WROW, HID), jnp.bfloat16),
        ],
        compiler_params=pltpu.CompilerParams(
            dimension_semantics=("parallel", "parallel")),
    )(q, k, v, W1.reshape(WCH, WROW, FFN), W2.reshape(WCH, WROW, HID))
    W1b = W1b.reshape(NE, HID, FFN)
    W2b = W2b.reshape(NE, FFN, HID)

    WoT = Wo.T.astype(jnp.bfloat16)
    out = pl.pallas_call(
        _block_moe_kernel,
        grid=(SEQ // ROWB,),
        in_specs=[
            pl.BlockSpec((HEADS, ROWB, DH), lambda i: (0, i, 0)),
            pl.BlockSpec((HID, HID), lambda i: (0, 0)),
            pl.BlockSpec((1, HID), lambda i: (0, 0)),
            pl.BlockSpec((ROWB, HID), lambda i: (i, 0)),
            pl.BlockSpec((1, HID), lambda i: (0, 0)),
            pl.BlockSpec((1, HID), lambda i: (0, 0)),
            pl.BlockSpec((HID, NE), lambda i: (0, 0)),
            pl.BlockSpec((1, NE), lambda i: (0, 0)),
            pl.BlockSpec((NE, HID, FFN), lambda i: (0, 0, 0)),
            pl.BlockSpec((NE, 1, FFN), lambda i: (0, 0, 0)),
            pl.BlockSpec((NE, FFN, HID), lambda i: (0, 0, 0)),
            pl.BlockSpec((NE, 1, HID), lambda i: (0, 0, 0)),
        ],
        out_specs=pl.BlockSpec((ROWB, HID), lambda i: (i, 0)),
        out_shape=jax.ShapeDtypeStruct((SEQ, HID), jnp.float32),
        scratch_shapes=[pltpu.VMEM((ROWB, HID), jnp.bfloat16)],
        compiler_params=pltpu.CompilerParams(
            dimension_semantics=("arbitrary",)),
    )(attn, WoT, bo.reshape(1, HID), xf, gamma2.reshape(1, HID),
      beta2.reshape(1, HID), Wr, br.reshape(1, NE),
      W1b, b1.reshape(NE, 1, FFN), W2b, b2.reshape(NE, 1, HID))

    return out.reshape(1, SEQ, HID)


# speedup vs baseline: 2.5892x; 1.0876x over previous
import functools

import jax
import jax.numpy as jnp
from jax.experimental import pallas as pl
from jax.experimental.pallas import tpu as pltpu

HID = 768
HEADS = 12
DH = 64
NE = 8
TOP2 = 2
FFN = 768
SEQ = 2048
ROWB = 256
QB = 1024
HP = 4
WCH = (HEADS // HP) * (SEQ // QB)
WROW = NE * HID // WCH
QSCALE = (DH ** -0.5) * 1.4426950408889634


def _ln_qkv_kernel(x_ref, w_ref, b_ref, g_ref, be_ref, q_ref, k_ref, v_ref):
    x = x_ref[...]
    m = jnp.mean(x, axis=-1, keepdims=True)
    v = jnp.mean(jnp.square(x - m), axis=-1, keepdims=True)
    xn = (x - m) * jax.lax.rsqrt(v + 1e-5) * g_ref[...] + be_ref[...]
    o = jnp.dot(xn, w_ref[...],
                preferred_element_type=jnp.float32) + b_ref[...]
    for p, oref in enumerate((q_ref, k_ref, v_ref)):
        for h in range(HEADS):
            c = p * HEADS + h
            sl = o[:, c * DH:(c + 1) * DH]
            if p == 0:
                sl = sl * QSCALE
            oref[h] = sl.astype(jnp.bfloat16)


def _attn_kernel(q_ref, k_ref, v_ref, w1_ref, w2_ref, o_ref, w1b_ref,
                 w2b_ref):
    w1b_ref[...] = w1_ref[...].astype(jnp.bfloat16)
    w2b_ref[...] = w2_ref[...].astype(jnp.bfloat16)
    for j in range(HP):
        q = q_ref[j]
        k = k_ref[j]
        v = v_ref[j]
        s = jax.lax.dot_general(q, k, (((1,), (1,)), ((), ())),
                                preferred_element_type=jnp.float32)
        sb = s.astype(jnp.bfloat16)
        m = jnp.max(sb, axis=-1, keepdims=True)
        pb = jnp.exp2((sb - m).astype(jnp.float32)).astype(jnp.bfloat16)
        l = jnp.sum(pb, axis=-1, keepdims=True, dtype=jnp.float32)
        o = jnp.dot(pb, v, preferred_element_type=jnp.float32)
        o_ref[j] = (o * (1.0 / l)).astype(jnp.bfloat16)


def _block_moe_kernel(a_ref, wo_ref, bo_ref, res_ref, g2_ref, b2ln_ref,
                      wr_ref, br_ref, w1_ref, b1_ref, w2_ref, b2_ref,
                      o_ref, acc_ref):
    for h in range(HEADS):
        acc_ref[:, h * DH:(h + 1) * DH] = a_ref[h]
    h1 = jnp.dot(acc_ref[...], wo_ref[...],
                 preferred_element_type=jnp.float32)
    h1 = h1 + bo_ref[...] + res_ref[...]
    m = jnp.mean(h1, axis=-1, keepdims=True)
    va = jnp.mean(jnp.square(h1 - m), axis=-1, keepdims=True)
    t = (h1 - m) * jax.lax.rsqrt(va + 1e-5) * g2_ref[...] + b2ln_ref[...]
    logits = jnp.dot(t, wr_ref[...],
                     preferred_element_type=jnp.float32) + br_ref[...]
    lm = jnp.max(logits, axis=-1, keepdims=True)
    pe = jnp.exp(logits - lm)
    probs = pe / jnp.sum(pe, axis=-1, keepdims=True)
    v1 = jnp.max(probs, axis=-1, keepdims=True)
    cols = jax.lax.broadcasted_iota(jnp.int32, probs.shape, 1)
    i1 = jnp.argmax(probs, axis=-1)
    masked = jnp.where(cols == i1[:, None], -jnp.inf, probs)
    v2 = jnp.max(masked, axis=-1, keepdims=True)
    i2 = jnp.argmax(masked, axis=-1)
    tot = v1 + v2
    gates = jnp.where(cols == i1[:, None], v1 / tot,
                      jnp.where(cols == i2[:, None], v2 / tot, 0.0))
    out = h1
    tb = t.astype(jnp.bfloat16)
    for e in range(NE):
        hm = jnp.dot(tb, w1_ref[e],
                     preferred_element_type=jnp.float32) + b1_ref[e]
        hm = jax.nn.gelu(hm)
        y = jnp.dot(hm.astype(jnp.bfloat16), w2_ref[e],
                    preferred_element_type=jnp.float32) + b2_ref[e]
        out = out + y * gates[:, e:e + 1]
    o_ref[...] = out


def kernel(x, gamma1, beta1, Wqkv, bqkv, Wo, bo, gamma2, beta2, Wr, br,
           W1, b1, W2, b2):
    xf = x.reshape(SEQ, HID)

    q, k, v = pl.pallas_call(
        _ln_qkv_kernel,
        grid=(SEQ // ROWB,),
        in_specs=[
            pl.BlockSpec((ROWB, HID), lambda i: (i, 0)),
            pl.BlockSpec((HID, 3 * HID), lambda i: (0, 0)),
            pl.BlockSpec((1, 3 * HID), lambda i: (0, 0)),
            pl.BlockSpec((1, HID), lambda i: (0, 0)),
            pl.BlockSpec((1, HID), lambda i: (0, 0)),
        ],
        out_specs=[
            pl.BlockSpec((HEADS, ROWB, DH), lambda i: (0, i, 0)),
            pl.BlockSpec((HEADS, ROWB, DH), lambda i: (0, i, 0)),
            pl.BlockSpec((HEADS, ROWB, DH), lambda i: (0, i, 0)),
        ],
        out_shape=[jax.ShapeDtypeStruct((HEADS, SEQ, DH), jnp.bfloat16)] * 3,
        compiler_params=pltpu.CompilerParams(
            dimension_semantics=("parallel",)),
    )(xf, Wqkv.T, bqkv.reshape(1, 3 * HID),
      gamma1.reshape(1, HID), beta1.reshape(1, HID))

    nq = SEQ // QB
    attn, W1b, W2b = pl.pallas_call(
        _attn_kernel,
        grid=(HEADS // HP, nq),
        in_specs=[
            pl.BlockSpec((HP, QB, DH), lambda h, i: (h, i, 0)),
            pl.BlockSpec((HP, SEQ, DH), lambda h, i: (h, 0, 0)),
            pl.BlockSpec((HP, SEQ, DH), lambda h, i: (h, 0, 0)),
            pl.BlockSpec((1, WROW, FFN), lambda h, i: (h * nq + i, 0, 0)),
            pl.BlockSpec((1, WROW, HID), lambda h, i: (h * nq + i, 0, 0)),
        ],
        out_specs=[
            pl.BlockSpec((HP, QB, DH), lambda h, i: (h, i, 0)),
            pl.BlockSpec((1, WROW, FFN), lambda h, i: (h * nq + i, 0, 0)),
            pl.BlockSpec((1, WROW, HID), lambda h, i: (h * nq + i, 0, 0)),
        ],
        out_shape=[
            jax.ShapeDtypeStruct((HEADS, SEQ, DH), jnp.bfloat16),
            jax.ShapeDtypeStruct((WCH, WROW, FFN), jnp.bfloat16),
            jax.ShapeDtypeStruct((WCH, WROW, HID), jnp.bfloat16),
        ],
        compiler_params=pltpu.CompilerParams(
            dimension_semantics=("parallel", "parallel")),
    )(q, k, v, W1.reshape(WCH, WROW, FFN), W2.reshape(WCH, WROW, HID))
    W1b = W1b.reshape(NE, HID, FFN)
    W2b = W2b.reshape(NE, FFN, HID)

    WoT = Wo.T.astype(jnp.bfloat16)
    out = pl.pallas_call(
        _block_moe_kernel,
        grid=(SEQ // ROWB,),
        in_specs=[
            pl.BlockSpec((HEADS, ROWB, DH), lambda i: (0, i, 0)),
            pl.BlockSpec((HID, HID), lambda i: (0, 0)),
            pl.BlockSpec((1, HID), lambda i: (0, 0)),
            pl.BlockSpec((ROWB, HID), lambda i: (i, 0)),
            pl.BlockSpec((1, HID), lambda i: (0, 0)),
            pl.BlockSpec((1, HID), lambda i: (0, 0)),
            pl.BlockSpec((HID, NE), lambda i: (0, 0)),
            pl.BlockSpec((1, NE), lambda i: (0, 0)),
            pl.BlockSpec((NE, HID, FFN), lambda i: (0, 0, 0)),
            pl.BlockSpec((NE, 1, FFN), lambda i: (0, 0, 0)),
            pl.BlockSpec((NE, FFN, HID), lambda i: (0, 0, 0)),
            pl.BlockSpec((NE, 1, HID), lambda i: (0, 0, 0)),
        ],
        out_specs=pl.BlockSpec((ROWB, HID), lambda i: (i, 0)),
        out_shape=jax.ShapeDtypeStruct((SEQ, HID), jnp.float32),
        scratch_shapes=[pltpu.VMEM((ROWB, HID), jnp.bfloat16)],
        compiler_params=pltpu.CompilerParams(
            dimension_semantics=("arbitrary",)),
    )(attn, WoT, bo.reshape(1, HID), xf, gamma2.reshape(1, HID),
      beta2.reshape(1, HID), Wr, br.reshape(1, NE),
      W1b, b1.reshape(NE, 1, FFN), W2b, b2.reshape(NE, 1, HID))

    return out.reshape(1, SEQ, HID)
